# baseline (speedup 1.0000x reference)
import functools

import jax
import jax.numpy as jnp
from jax import lax
from jax.experimental import pallas as pl
from jax.experimental.pallas import tpu as pltpu
from jax.experimental.pallas import tpu_sc as plsc

N_EDGES = 800000
EMBED_DIM = 64
NUM_CORES = 2
NUM_SUBCORES = 16
NW = NUM_CORES * NUM_SUBCORES
PER_W = N_EDGES // NW
CHUNK = 1000
NCHUNK = PER_W // CHUNK


def _body(w_hbm, idx_hbm, out_hbm, idx_v, rows_v, gsem):
    cid = lax.axis_index("c")
    sid = lax.axis_index("s")
    wid = sid * NUM_CORES + cid
    base = wid * PER_W

    pltpu.sync_copy(idx_hbm.at[pl.ds(base, PER_W)], idx_v)

    def chunk_step(j, carry):
        off = j * CHUNK
        pltpu.async_copy(
            w_hbm.at[idx_v.at[pl.ds(off, CHUNK)]], rows_v, gsem
        ).wait()
        pltpu.sync_copy(rows_v, out_hbm.at[pl.ds(base + off, CHUNK)])
        return carry

    lax.fori_loop(0, NCHUNK, chunk_step, 0)


def kernel(edge_attr, weight):
    idx = edge_attr.astype(jnp.int32)
    mesh = plsc.VectorSubcoreMesh(core_axis_name="c", subcore_axis_name="s")
    k = pl.kernel(
        _body,
        out_type=jax.ShapeDtypeStruct((N_EDGES, EMBED_DIM), jnp.float32),
        mesh=mesh,
        scratch_types=[
            pltpu.VMEM((PER_W,), jnp.int32),
            pltpu.VMEM((CHUNK, EMBED_DIM), jnp.float32),
            pltpu.SemaphoreType.DMA,
        ],
        compiler_params=pltpu.CompilerParams(use_tc_tiling_on_sc=False),
    )
    return k(weight, idx)


# speedup vs baseline: 3.5297x; 3.5297x over previous
import jax
import jax.numpy as jnp
from jax import lax
from jax.experimental import pallas as pl
from jax.experimental.pallas import tpu as pltpu
from jax.experimental.pallas import tpu_sc as plsc

N_EDGES = 800000
EMBED_DIM = 64
NUM_CORES = 2
NUM_SUBCORES = 16
NW = NUM_CORES * NUM_SUBCORES
PER_W = N_EDGES // NW
CHUNK = 200
NCHUNK = PER_W // CHUNK
NGROUP = 13


def _body(w_hbm, idx_hbm, out_hbm, idx_v, table_v, buf0, buf1, sem0, sem1):
    cid = lax.axis_index("c")
    sid = lax.axis_index("s")
    wid = sid * NUM_CORES + cid
    base = wid * PER_W

    pltpu.sync_copy(w_hbm, table_v)
    pltpu.sync_copy(idx_hbm.at[pl.ds(base, PER_W)], idx_v)

    iota = lax.iota(jnp.int32, 16)

    def compute_chunk(j, buf):
        coff = j * CHUNK

        def group_step(g, carry):
            goff = lax.min(g * 16, CHUNK - 16)
            idx16 = idx_v[pl.ds(coff + goff, 16)]
            row16 = iota + goff
            for c in range(EMBED_DIM):
                cvec = jnp.full((16,), c, jnp.int32)
                vals = plsc.load_gather(table_v, [idx16, cvec])
                plsc.store_scatter(buf, [row16, cvec], vals)
            return carry

        lax.fori_loop(0, NGROUP, group_step, 0)

    def start_write(j, buf, sem):
        pltpu.async_copy(buf, out_hbm.at[pl.ds(base + j * CHUNK, CHUNK)], sem)

    def wait_write(buf, sem):
        pltpu.make_async_copy(buf, out_hbm.at[pl.ds(base, CHUNK)], sem).wait()

    compute_chunk(0, buf0)
    start_write(0, buf0, sem0)

    def pair_step(t, carry):
        j1 = 2 * t + 1
        j2 = 2 * t + 2
        compute_chunk(j1, buf1)
        start_write(j1, buf1, sem1)
        wait_write(buf0, sem0)
        compute_chunk(j2, buf0)
        start_write(j2, buf0, sem0)
        wait_write(buf1, sem1)
        return carry

    lax.fori_loop(0, (NCHUNK - 1) // 2, pair_step, 0)
    wait_write(buf0, sem0)


def kernel(edge_attr, weight):
    idx = edge_attr.astype(jnp.int32)
    mesh = plsc.VectorSubcoreMesh(core_axis_name="c", subcore_axis_name="s")
    k = pl.kernel(
        _body,
        out_type=jax.ShapeDtypeStruct((N_EDGES, EMBED_DIM), jnp.float32),
        mesh=mesh,
        scratch_types=[
            pltpu.VMEM((PER_W,), jnp.int32),
            pltpu.VMEM((4, EMBED_DIM), jnp.float32),
            pltpu.VMEM((CHUNK, EMBED_DIM), jnp.float32),
            pltpu.VMEM((CHUNK, EMBED_DIM), jnp.float32),
            pltpu.SemaphoreType.DMA,
            pltpu.SemaphoreType.DMA,
        ],
        compiler_params=pltpu.CompilerParams(
            use_tc_tiling_on_sc=False, needs_layout_passes=False
        ),
    )
    return k(weight, idx)


# speedup vs baseline: 13.9365x; 3.9483x over previous
import jax
import jax.numpy as jnp
from jax import lax
from jax.experimental import pallas as pl
from jax.experimental.pallas import tpu as pltpu
from jax.experimental.pallas import tpu_sc as plsc

N_EDGES = 800000
EMBED_DIM = 64
NUM_CORES = 2
NUM_SUBCORES = 16
NW = NUM_CORES * NUM_SUBCORES
PER_W = N_EDGES // NW
CHUNK = 200
NCHUNK = PER_W // CHUNK
NBUF = 4


def _body(w_hbm, idx_hbm, out_hbm, idx_v, table_s,
          b0, b1, b2, b3, g0, g1, g2, g3, w0, w1, w2, w3):
    cid = lax.axis_index("c")
    sid = lax.axis_index("s")
    wid = sid * NUM_CORES + cid
    base = wid * PER_W

    bufs = (b0, b1, b2, b3)
    gsems = (g0, g1, g2, g3)
    wsems = (w0, w1, w2, w3)

    @pl.when(sid == 0)
    def _stage_table():
        pltpu.sync_copy(w_hbm, table_s)

    pltpu.sync_copy(idx_hbm.at[pl.ds(base, PER_W)], idx_v)
    plsc.subcore_barrier()

    def start_gather(j, b):
        pltpu.async_copy(
            table_s.at[idx_v.at[pl.ds(j * CHUNK, CHUNK)]], bufs[b], gsems[b]
        )

    def wait_gather(b):
        pltpu.make_async_copy(
            table_s.at[idx_v.at[pl.ds(0, CHUNK)]], bufs[b], gsems[b]
        ).wait()

    def start_write(j, b):
        pltpu.async_copy(
            bufs[b], out_hbm.at[pl.ds(base + j * CHUNK, CHUNK)], wsems[b]
        )

    def wait_write(b):
        pltpu.make_async_copy(
            bufs[b], out_hbm.at[pl.ds(base, CHUNK)], wsems[b]
        ).wait()

    for b in range(NBUF):
        start_gather(b, b)

    def step(t, carry):
        for b in range(NBUF):
            wait_gather(b)
            start_write(NBUF * t + b, b)
        for b in range(NBUF):
            wait_write(b)
            start_gather(NBUF * t + NBUF + b, b)
        return carry

    lax.fori_loop(0, (NCHUNK - NBUF - 1) // NBUF, step, 0)

    for b in range(NBUF):
        wait_gather(b)
        start_write(120 + b, b)
    wait_write(0)
    start_gather(NCHUNK - 1, 0)
    wait_gather(0)
    start_write(NCHUNK - 1, 0)
    wait_write(0)
    for b in range(1, NBUF):
        wait_write(b)


def kernel(edge_attr, weight):
    idx = edge_attr.astype(jnp.int32)
    mesh = plsc.VectorSubcoreMesh(core_axis_name="c", subcore_axis_name="s")
    k = pl.kernel(
        _body,
        out_type=jax.ShapeDtypeStruct((N_EDGES, EMBED_DIM), jnp.float32),
        mesh=mesh,
        scratch_types=[
            pltpu.VMEM((PER_W,), jnp.int32),
            pltpu.VMEM_SHARED((4, EMBED_DIM), jnp.float32),
            pltpu.VMEM((CHUNK, EMBED_DIM), jnp.float32),
            pltpu.VMEM((CHUNK, EMBED_DIM), jnp.float32),
            pltpu.VMEM((CHUNK, EMBED_DIM), jnp.float32),
            pltpu.VMEM((CHUNK, EMBED_DIM), jnp.float32),
            pltpu.SemaphoreType.DMA,
            pltpu.SemaphoreType.DMA,
            pltpu.SemaphoreType.DMA,
            pltpu.SemaphoreType.DMA,
            pltpu.SemaphoreType.DMA,
            pltpu.SemaphoreType.DMA,
            pltpu.SemaphoreType.DMA,
            pltpu.SemaphoreType.DMA,
        ],
        compiler_params=pltpu.CompilerParams(
            use_tc_tiling_on_sc=False, needs_layout_passes=False
        ),
    )
    return k(weight, idx)


# speedup vs baseline: 18.6249x; 1.3364x over previous
import jax
import jax.numpy as jnp
from jax import lax
from jax.experimental import pallas as pl
from jax.experimental.pallas import tpu as pltpu
from jax.experimental.pallas import tpu_sc as plsc

N_EDGES = 800000
EMBED_DIM = 64
NPAIR = N_EDGES // 2
NUM_CORES = 2
NUM_SUBCORES = 16
NW = NUM_CORES * NUM_SUBCORES
PER_W = NPAIR // NW
STAGE = 12504
CHUNK = 200
NCHUNK = 64
NBUF = 4

TR_ROWS = 3200
TR_NB = NPAIR // TR_ROWS


def _sc_body(w2_hbm, idx_hbm, out_hbm, idx_v, table_s,
             b0, b1, b2, b3, g0, g1, g2, g3, w0, w1, w2, w3):
    cid = lax.axis_index("c")
    sid = lax.axis_index("s")
    wid = sid * NUM_CORES + cid

    raw = wid * PER_W
    base = raw + (raw % 8)
    nraw = (wid + 1) * PER_W
    nbase = nraw + (nraw % 8)
    size = nbase - base
    base = pl.multiple_of(base, 8)

    bufs = (b0, b1, b2, b3)
    gsems = (g0, g1, g2, g3)
    wsems = (w0, w1, w2, w3)

    @pl.when(sid == 0)
    def _stage_table():
        pltpu.sync_copy(w2_hbm, table_s)

    pltpu.sync_copy(idx_hbm.at[pl.ds(base, STAGE)], idx_v)
    plsc.subcore_barrier()

    def chunk_off(j):
        return pl.multiple_of(lax.min(j * CHUNK, size - CHUNK), 8)

    def start_gather(j, b):
        pltpu.async_copy(
            table_s.at[idx_v.at[pl.ds(chunk_off(j), CHUNK)]], bufs[b], gsems[b]
        )

    def wait_gather(b):
        pltpu.make_async_copy(
            table_s.at[idx_v.at[pl.ds(0, CHUNK)]], bufs[b], gsems[b]
        ).wait()

    def start_write(j, b):
        pltpu.async_copy(
            bufs[b], out_hbm.at[pl.ds(base + chunk_off(j), CHUNK)], wsems[b]
        )

    def wait_write(b):
        pltpu.make_async_copy(
            bufs[b], out_hbm.at[pl.ds(base, CHUNK)], wsems[b]
        ).wait()

    for b in range(NBUF):
        start_gather(b, b)

    def step(t, carry):
        for b in range(NBUF):
            wait_gather(b)
            start_write(NBUF * t + b, b)
        for b in range(NBUF):
            wait_write(b)
            start_gather(NBUF * t + NBUF + b, b)
        return carry

    lax.fori_loop(0, (NCHUNK - NBUF) // NBUF, step, 0)

    for b in range(NBUF):
        wait_gather(b)
        start_write(NCHUNK - NBUF + b, b)
    for b in range(NBUF):
        wait_write(b)


def _sc_gather(pidx_padded, w2):
    mesh = plsc.VectorSubcoreMesh(core_axis_name="c", subcore_axis_name="s")
    k = pl.kernel(
        _sc_body,
        out_type=jax.ShapeDtypeStruct((NPAIR, 2 * EMBED_DIM), jnp.float32),
        mesh=mesh,
        scratch_types=[
            pltpu.VMEM((STAGE,), jnp.int32),
            pltpu.VMEM_SHARED((16, 2 * EMBED_DIM), jnp.float32),
            pltpu.VMEM((CHUNK, 2 * EMBED_DIM), jnp.float32),
            pltpu.VMEM((CHUNK, 2 * EMBED_DIM), jnp.float32),
            pltpu.VMEM((CHUNK, 2 * EMBED_DIM), jnp.float32),
            pltpu.VMEM((CHUNK, 2 * EMBED_DIM), jnp.float32),
            pltpu.SemaphoreType.DMA,
            pltpu.SemaphoreType.DMA,
            pltpu.SemaphoreType.DMA,
            pltpu.SemaphoreType.DMA,
            pltpu.SemaphoreType.DMA,
            pltpu.SemaphoreType.DMA,
            pltpu.SemaphoreType.DMA,
            pltpu.SemaphoreType.DMA,
        ],
        compiler_params=pltpu.CompilerParams(
            use_tc_tiling_on_sc=False, needs_layout_passes=False
        ),
    )
    return k(w2, pidx_padded)


def _tr_body(x_ref, y_ref):
    h = pl.program_id(1)
    eye = jax.lax.broadcasted_iota(jnp.int32, (EMBED_DIM, EMBED_DIM), 0)
    eyef = jnp.where(
        eye == jax.lax.broadcasted_iota(jnp.int32, (EMBED_DIM, EMBED_DIM), 1),
        1.0,
        0.0,
    ).astype(jnp.float32)

    def tr(xh):
        return jax.lax.dot_general(
            eyef, xh, (((1,), (1,)), ((), ())),
            preferred_element_type=jnp.float32,
        )

    y_ref[...] = jnp.where(
        h == 0, tr(x_ref[:, :EMBED_DIM]), tr(x_ref[:, EMBED_DIM:])
    )


def _tc_transpose(x):
    return pl.pallas_call(
        _tr_body,
        grid=(TR_NB, 2),
        in_specs=[pl.BlockSpec((TR_ROWS, 2 * EMBED_DIM), lambda r, h: (r, 0))],
        out_specs=pl.BlockSpec((EMBED_DIM, TR_ROWS), lambda r, h: (0, h * TR_NB + r)),
        out_shape=jax.ShapeDtypeStruct((EMBED_DIM, N_EDGES), jnp.float32),
    )(x)


def kernel(edge_attr, weight):
    idx = edge_attr.astype(jnp.int32)
    pidx = idx[:NPAIR] * 4 + idx[NPAIR:]
    pidx_padded = jnp.zeros((NPAIR + 128,), jnp.int32).at[:NPAIR].set(pidx)
    w2 = jnp.concatenate(
        [jnp.repeat(weight, 4, axis=0), jnp.tile(weight, (4, 1))], axis=1
    )
    x = _sc_gather(pidx_padded, w2)
    y = _tc_transpose(x)
    return y.T
